# baseline (speedup 1.0000x reference)
import functools

import jax
import jax.numpy as jnp
from jax import lax
from jax.experimental import pallas as pl
from jax.experimental.pallas import tpu as pltpu
from jax.experimental.pallas import tpu_sc as plsc

N_NODES = 50000
D = 128
NCHUNK = 4
CW = D // NCHUNK
NC, NS = 2, 16
G = 256
SW = 15
NSW = 5
GPT = G * SW * NSW
EPAD = GPT * NS
NPAD = 64
NACC = N_NODES + NPAD
ROWS_Z = NACC // NS
ROWS_W = N_NODES // NS
BM = 2000


def _mm_body(xu_ref, xi_ref, wsu_ref, wsi_ref, wt0_ref, wt1_ref,
             s_ref, t_ref):
    xu = xu_ref[...]
    xi = xi_ref[...]
    s_ref[0] = jnp.dot(
        xu, wsu_ref[...], preferred_element_type=jnp.float32).astype(jnp.bfloat16)
    s_ref[1] = jnp.dot(
        xi, wsi_ref[...], preferred_element_type=jnp.float32).astype(jnp.bfloat16)
    t_ref[0] = jnp.dot(xi, wt0_ref[...], preferred_element_type=jnp.float32)
    t_ref[1] = jnp.dot(xu, wt1_ref[...], preferred_element_type=jnp.float32)


def _epilogue_body(t_ref, s_ref, c_ref, out_ref):
    c0 = jnp.maximum(c_ref[0][:, 0:1], 1.0)
    c1 = jnp.maximum(c_ref[1][:, 0:1], 1.0)
    s0 = s_ref[0].astype(jnp.float32)
    s1 = s_ref[1].astype(jnp.float32)
    out_ref[0] = jnp.maximum(t_ref[1] + s1 / c1, 0.0)
    out_ref[1] = jnp.maximum(t_ref[0] + s0 / c0, 0.0)


def _sc_agg(s_flat, srcx_all, dst_all, sums_out, cnt_out,
            idx_sg, idx_dw, buf0, buf1, buf2, ones_v, acc, acc_cnt,
            gs0, gs1, gs2, ss0, ss1, ss2):
    cid = lax.axis_index("c")
    sid = lax.axis_index("s")
    bufs = (buf0, buf1, buf2)
    gsem = (gs0, gs1, gs2)
    ssem = (ss0, ss1, ss2)
    ebase = sid * GPT
    wbase = sid * ROWS_W

    def fill_bf(ref, val):
        def body(i, carry):
            ref[i, pl.ds(0, 32)] = jnp.full((32,), val, jnp.bfloat16)
            return carry
        lax.fori_loop(0, G, body, 0)

    def fill_f32(ref, val):
        def body(i, carry):
            ref[i, pl.ds(0, 16)] = jnp.full((16,), val, jnp.float32)
            return carry
        lax.fori_loop(0, G, body, 0)

    def zero_acc(dst, zbuf):
        base = sid * ROWS_Z
        nfull = ROWS_Z // G

        def body(j, carry):
            pltpu.async_copy(zbuf, dst.at[pl.ds(base + G * j, G), :], ss0)
            return carry
        lax.fori_loop(0, nfull, body, 0)
        rem = ROWS_Z - nfull * G
        pltpu.sync_copy(zbuf.at[pl.ds(0, rem), :],
                        dst.at[pl.ds(base + nfull * G, rem), :])

        def drain(j, carry):
            pltpu.make_async_copy(zbuf, dst.at[pl.ds(base, G), :], ss0).wait()
            return carry
        lax.fori_loop(0, nfull, drain, 0)

    def drain_scatter(s):
        pltpu.make_async_copy(
            bufs[s], acc.at[idx_dw.at[pl.ds(0, G)]], ssem[s]).wait()

    def chunk_pipeline(r, c):
        tab = s_flat.at[r]

        def issue_gather(h, u, s):
            pltpu.sync_copy(
                srcx_all.at[r, c, pl.ds(ebase + (h * SW + u) * G, G)],
                idx_sg.at[s])
            return pltpu.async_copy(tab.at[idx_sg.at[s]], bufs[s], gsem[s])

        def sw_body(h, carry):
            @pl.when(h > 0)
            def _():
                for s in range(3):
                    drain_scatter(s)
            dwd = pltpu.async_copy(
                dst_all.at[r, pl.ds(ebase + h * (G * SW), G * SW)], idx_dw,
                ss0)
            gd = [None] * SW
            for s in range(3):
                gd[s] = issue_gather(h, s, s)
            dwd.wait()
            for u in range(SW):
                s = u % 3
                gd[u].wait()
                sd = pltpu.async_copy(
                    bufs[s], acc.at[idx_dw.at[pl.ds(u * G, G)]], ssem[s],
                    add=True)
                if u + 3 < SW:
                    sd.wait()
                    gd[u + 3] = issue_gather(h, u + 3, s)
            return carry
        lax.fori_loop(0, NSW, sw_body, 0)
        for s in range(3):
            drain_scatter(s)

    for r in range(2):
        for p in range(2):
            fill_bf(buf0, 0.0)
            zero_acc(acc, buf0)
            plsc.subcore_barrier()
            for cc in range(NC):
                @pl.when(cid == cc)
                def _(r=r, c=NC * cc + p):
                    chunk_pipeline(r, c)
            plsc.subcore_barrier()
            for cc in range(NC):
                @pl.when(cid == cc)
                def _(r=r, c=NC * cc + p):
                    pltpu.sync_copy(
                        acc.at[pl.ds(wbase, ROWS_W), :],
                        sums_out.at[r, pl.ds(wbase, ROWS_W), pl.ds(CW * c, CW)])
            plsc.subcore_barrier()

    fill_f32(ones_v, 0.0)
    zero_acc(acc_cnt, ones_v)
    fill_f32(ones_v, 1.0)
    plsc.subcore_barrier()
    for cc in range(NC):
        @pl.when(cid == cc)
        def _(cc=cc):
            def sw_body(h, carry):
                pltpu.sync_copy(
                    dst_all.at[cc, pl.ds(ebase + h * (G * SW), G * SW)],
                    idx_dw)
                descs = [
                    pltpu.async_copy(
                        ones_v, acc_cnt.at[idx_dw.at[pl.ds(u * G, G)]], ss0,
                        add=True)
                    for u in range(SW)
                ]
                for d in descs:
                    d.wait()
                return carry
            lax.fori_loop(0, NSW, sw_body, 0)
    plsc.subcore_barrier()
    for cc in range(NC):
        @pl.when(cid == cc)
        def _(cc=cc):
            pltpu.sync_copy(acc_cnt.at[pl.ds(wbase, ROWS_W), :],
                            cnt_out.at[cc, pl.ds(wbase, ROWS_W), pl.ds(0, 16)])


@functools.cache
def _sc_agg_call():
    return pl.kernel(
        _sc_agg,
        out_type=(jax.ShapeDtypeStruct((2, N_NODES, D), jnp.bfloat16),
                  jax.ShapeDtypeStruct((2, N_NODES, D), jnp.float32)),
        mesh=plsc.VectorSubcoreMesh(core_axis_name="c", subcore_axis_name="s",
                                    num_cores=NC, num_subcores=NS),
        scratch_types=[
            pltpu.VMEM((3, G), jnp.int32),
            pltpu.VMEM((G * SW,), jnp.int32),
            pltpu.VMEM((G, CW), jnp.bfloat16),
            pltpu.VMEM((G, CW), jnp.bfloat16),
            pltpu.VMEM((G, CW), jnp.bfloat16),
            pltpu.VMEM((G, 16), jnp.float32),
            pltpu.VMEM_SHARED((NACC, CW), jnp.bfloat16),
            pltpu.VMEM_SHARED((NACC, 16), jnp.float32),
            pltpu.SemaphoreType.DMA,
            pltpu.SemaphoreType.DMA,
            pltpu.SemaphoreType.DMA,
            pltpu.SemaphoreType.DMA,
            pltpu.SemaphoreType.DMA,
            pltpu.SemaphoreType.DMA,
        ],
        compiler_params=pltpu.CompilerParams(use_tc_tiling_on_sc=False),
    )


def _prep_edges(edge):
    pad = EPAD - edge.shape[1]
    padr = jnp.arange(pad, dtype=jnp.int32)
    src = jnp.concatenate([edge[0], padr % 2048])
    srcx = jnp.stack([src * NCHUNK + c for c in range(NCHUNK)])
    dst = jnp.concatenate([edge[1], N_NODES + (padr % NPAD)])
    return srcx, dst


def kernel(x_user, x_item, edge_rates, edge_rated_by,
           W_rates_src, W_rates_tgt, W_rb_src, W_rb_tgt):
    n_blocks = N_NODES // BM

    s_all, t_all = pl.pallas_call(
        _mm_body,
        grid=(n_blocks,),
        in_specs=[
            pl.BlockSpec((BM, D), lambda i: (i, 0)),
            pl.BlockSpec((BM, D), lambda i: (i, 0)),
            pl.BlockSpec((D, D), lambda i: (0, 0)),
            pl.BlockSpec((D, D), lambda i: (0, 0)),
            pl.BlockSpec((D, D), lambda i: (0, 0)),
            pl.BlockSpec((D, D), lambda i: (0, 0)),
        ],
        out_specs=[
            pl.BlockSpec((2, BM, D), lambda i: (0, i, 0)),
            pl.BlockSpec((2, BM, D), lambda i: (0, i, 0)),
        ],
        out_shape=[
            jax.ShapeDtypeStruct((2, N_NODES, D), jnp.bfloat16),
            jax.ShapeDtypeStruct((2, N_NODES, D), jnp.float32),
        ],
    )(x_user, x_item, W_rates_src, W_rb_src, W_rates_tgt, W_rb_tgt)

    s_flat = s_all.reshape(2, NCHUNK * N_NODES, CW)

    srcx0, dst0 = _prep_edges(edge_rates)
    srcx1, dst1 = _prep_edges(edge_rated_by)
    srcx_all = jnp.stack([srcx0, srcx1])
    dst_all = jnp.stack([dst0, dst1])

    sums, cnt = _sc_agg_call()(s_flat, srcx_all, dst_all)

    out = pl.pallas_call(
        _epilogue_body,
        grid=(n_blocks,),
        in_specs=[
            pl.BlockSpec((2, BM, D), lambda i: (0, i, 0)),
            pl.BlockSpec((2, BM, D), lambda i: (0, i, 0)),
            pl.BlockSpec((2, BM, D), lambda i: (0, i, 0)),
        ],
        out_specs=pl.BlockSpec((2, BM, D), lambda i: (0, i, 0)),
        out_shape=jax.ShapeDtypeStruct((2, N_NODES, D), jnp.float32),
    )(t_all, sums, cnt)

    return out


# speedup vs baseline: 1.2145x; 1.2145x over previous
import functools

import jax
import jax.numpy as jnp
from jax import lax
from jax.experimental import pallas as pl
from jax.experimental.pallas import tpu as pltpu
from jax.experimental.pallas import tpu_sc as plsc

N_NODES = 50000
D = 128
NCHUNK = 4
CW = D // NCHUNK
NC, NS = 2, 16
G = 256
SW = 15
NSW = 5
GPT = G * SW * NSW
EPAD = GPT * NS
NPAD = 64
NACC = N_NODES + NPAD
ROWS_Z = NACC // NS
ROWS_W = N_NODES // NS
BM = 2000


def _mm_body(xu_ref, xi_ref, wsu_ref, wsi_ref, wt0_ref, wt1_ref,
             s_ref, t_ref):
    xu = xu_ref[...]
    xi = xi_ref[...]
    s_ref[0] = jnp.dot(xu, wsu_ref[...], preferred_element_type=jnp.float32)
    s_ref[1] = jnp.dot(xi, wsi_ref[...], preferred_element_type=jnp.float32)
    t_ref[0] = jnp.dot(xi, wt0_ref[...], preferred_element_type=jnp.float32)
    t_ref[1] = jnp.dot(xu, wt1_ref[...], preferred_element_type=jnp.float32)


def _epilogue_body(t_ref, s_ref, c_ref, out_ref):
    c0 = jnp.maximum(c_ref[0][:, 0:1], 1.0)
    c1 = jnp.maximum(c_ref[1][:, 0:1], 1.0)
    out_ref[0] = jnp.maximum(t_ref[1] + s_ref[1] / c1, 0.0)
    out_ref[1] = jnp.maximum(t_ref[0] + s_ref[0] / c0, 0.0)


def _sc_agg(s_flat, srcx_all, dst_all, sums_out, cnt_out,
            idx_sg, idx_dw, buf0, buf1, buf2, acc,
            gs0, gs1, gs2, ss0, ss1, ss2):
    cid = lax.axis_index("c")
    sid = lax.axis_index("s")
    bufs = (buf0, buf1, buf2)
    gsem = (gs0, gs1, gs2)
    ssem = (ss0, ss1, ss2)
    ebase = sid * GPT
    wbase = sid * ROWS_W

    def fill(ref, val):
        def body(i, carry):
            ref[i, pl.ds(0, 16)] = jnp.full((16,), val, jnp.float32)
            ref[i, pl.ds(16, 16)] = jnp.full((16,), val, jnp.float32)
            return carry
        lax.fori_loop(0, G, body, 0)

    def zero_acc(zbuf):
        base = sid * ROWS_Z
        nfull = ROWS_Z // G

        def body(j, carry):
            pltpu.async_copy(zbuf, acc.at[pl.ds(base + G * j, G), :], ss0)
            return carry
        lax.fori_loop(0, nfull, body, 0)
        rem = ROWS_Z - nfull * G
        pltpu.sync_copy(zbuf.at[pl.ds(0, rem), :],
                        acc.at[pl.ds(base + nfull * G, rem), :])

        def drain(j, carry):
            pltpu.make_async_copy(zbuf, acc.at[pl.ds(base, G), :], ss0).wait()
            return carry
        lax.fori_loop(0, nfull, drain, 0)

    def drain_scatter(s):
        pltpu.make_async_copy(
            bufs[s], acc.at[idx_dw.at[pl.ds(0, G)]], ssem[s]).wait()

    def chunk_pipeline(r, c):
        tab = s_flat.at[r]

        def issue_gather(h, u, s):
            pltpu.sync_copy(
                srcx_all.at[r, c, pl.ds(ebase + (h * SW + u) * G, G)],
                idx_sg.at[s])
            return pltpu.async_copy(tab.at[idx_sg.at[s]], bufs[s], gsem[s])

        def sw_body(h, carry):
            @pl.when(h > 0)
            def _():
                for s in range(3):
                    drain_scatter(s)
            dwd = pltpu.async_copy(
                dst_all.at[r, pl.ds(ebase + h * (G * SW), G * SW)], idx_dw,
                ss0)
            gd = [None] * SW
            for s in range(3):
                gd[s] = issue_gather(h, s, s)
            dwd.wait()
            for u in range(SW):
                s = u % 3
                gd[u].wait()
                sd = pltpu.async_copy(
                    bufs[s], acc.at[idx_dw.at[pl.ds(u * G, G)]], ssem[s],
                    add=True)
                if u + 3 < SW:
                    sd.wait()
                    gd[u + 3] = issue_gather(h, u + 3, s)
            return carry
        lax.fori_loop(0, NSW, sw_body, 0)
        for s in range(3):
            drain_scatter(s)

    for r in range(2):
        for p in range(2):
            fill(buf0, 0.0)
            zero_acc(buf0)
            plsc.subcore_barrier()
            for cc in range(NC):
                @pl.when(cid == cc)
                def _(r=r, c=NC * cc + p):
                    chunk_pipeline(r, c)
            plsc.subcore_barrier()
            for cc in range(NC):
                @pl.when(cid == cc)
                def _(r=r, c=NC * cc + p):
                    pltpu.sync_copy(
                        acc.at[pl.ds(wbase, ROWS_W), :],
                        sums_out.at[r, pl.ds(wbase, ROWS_W), pl.ds(CW * c, CW)])
            plsc.subcore_barrier()

    fill(buf1, 0.0)
    zero_acc(buf1)
    fill(buf0, 1.0)
    plsc.subcore_barrier()
    for cc in range(NC):
        @pl.when(cid == cc)
        def _(cc=cc):
            def sw_body(h, carry):
                pltpu.sync_copy(
                    dst_all.at[cc, pl.ds(ebase + h * (G * SW), G * SW)],
                    idx_dw)
                descs = [
                    pltpu.async_copy(
                        buf0, acc.at[idx_dw.at[pl.ds(u * G, G)]], ss0,
                        add=True)
                    for u in range(SW)
                ]
                for d in descs:
                    d.wait()
                return carry
            lax.fori_loop(0, NSW, sw_body, 0)
    plsc.subcore_barrier()
    for cc in range(NC):
        @pl.when(cid == cc)
        def _(cc=cc):
            pltpu.sync_copy(acc.at[pl.ds(wbase, ROWS_W), :],
                            cnt_out.at[cc, pl.ds(wbase, ROWS_W), pl.ds(0, CW)])


@functools.cache
def _sc_agg_call():
    return pl.kernel(
        _sc_agg,
        out_type=(jax.ShapeDtypeStruct((2, N_NODES, D), jnp.float32),
                  jax.ShapeDtypeStruct((2, N_NODES, D), jnp.float32)),
        mesh=plsc.VectorSubcoreMesh(core_axis_name="c", subcore_axis_name="s",
                                    num_cores=NC, num_subcores=NS),
        scratch_types=[
            pltpu.VMEM((3, G), jnp.int32),
            pltpu.VMEM((G * SW,), jnp.int32),
            pltpu.VMEM((G, CW), jnp.float32),
            pltpu.VMEM((G, CW), jnp.float32),
            pltpu.VMEM((G, CW), jnp.float32),
            pltpu.VMEM_SHARED((NACC, CW), jnp.float32),
            pltpu.SemaphoreType.DMA,
            pltpu.SemaphoreType.DMA,
            pltpu.SemaphoreType.DMA,
            pltpu.SemaphoreType.DMA,
            pltpu.SemaphoreType.DMA,
            pltpu.SemaphoreType.DMA,
        ],
        compiler_params=pltpu.CompilerParams(use_tc_tiling_on_sc=False),
    )


def _prep_edges(edge):
    pad = EPAD - edge.shape[1]
    padr = jnp.arange(pad, dtype=jnp.int32)
    src = jnp.concatenate([edge[0], padr % 2048])
    srcx = jnp.stack([src * NCHUNK + c for c in range(NCHUNK)])
    dst = jnp.concatenate([edge[1], N_NODES + (padr % NPAD)])
    return srcx, dst


def kernel(x_user, x_item, edge_rates, edge_rated_by,
           W_rates_src, W_rates_tgt, W_rb_src, W_rb_tgt):
    n_blocks = N_NODES // BM

    s_all, t_all = pl.pallas_call(
        _mm_body,
        grid=(n_blocks,),
        in_specs=[
            pl.BlockSpec((BM, D), lambda i: (i, 0)),
            pl.BlockSpec((BM, D), lambda i: (i, 0)),
            pl.BlockSpec((D, D), lambda i: (0, 0)),
            pl.BlockSpec((D, D), lambda i: (0, 0)),
            pl.BlockSpec((D, D), lambda i: (0, 0)),
            pl.BlockSpec((D, D), lambda i: (0, 0)),
        ],
        out_specs=[
            pl.BlockSpec((2, BM, D), lambda i: (0, i, 0)),
            pl.BlockSpec((2, BM, D), lambda i: (0, i, 0)),
        ],
        out_shape=[
            jax.ShapeDtypeStruct((2, N_NODES, D), jnp.float32),
            jax.ShapeDtypeStruct((2, N_NODES, D), jnp.float32),
        ],
    )(x_user, x_item, W_rates_src, W_rb_src, W_rates_tgt, W_rb_tgt)

    s_flat = s_all.reshape(2, NCHUNK * N_NODES, CW)

    srcx0, dst0 = _prep_edges(edge_rates)
    srcx1, dst1 = _prep_edges(edge_rated_by)
    srcx_all = jnp.stack([srcx0, srcx1])
    dst_all = jnp.stack([dst0, dst1])

    sums, cnt = _sc_agg_call()(s_flat, srcx_all, dst_all)

    out = pl.pallas_call(
        _epilogue_body,
        grid=(n_blocks,),
        in_specs=[
            pl.BlockSpec((2, BM, D), lambda i: (0, i, 0)),
            pl.BlockSpec((2, BM, D), lambda i: (0, i, 0)),
            pl.BlockSpec((2, BM, D), lambda i: (0, i, 0)),
        ],
        out_specs=pl.BlockSpec((2, BM, D), lambda i: (0, i, 0)),
        out_shape=jax.ShapeDtypeStruct((2, N_NODES, D), jnp.float32),
    )(t_all, sums, cnt)

    return out


# speedup vs baseline: 1.2739x; 1.0488x over previous
import functools

import jax
import jax.numpy as jnp
from jax import lax
from jax.experimental import pallas as pl
from jax.experimental.pallas import tpu as pltpu
from jax.experimental.pallas import tpu_sc as plsc

N_NODES = 50000
D = 128
NCHUNK = 4
CW = D // NCHUNK
NC, NS = 2, 16
G = 256
SW = 15
NSW = 5
GPT = G * SW * NSW
EPAD = GPT * NS
NPAD = 64
NACC = N_NODES + NPAD
ROWS_Z = NACC // NS
ROWS_W = N_NODES // NS
BM = 2000


def _mm_body(xu_ref, xi_ref, wsu_ref, wsi_ref, wt0_ref, wt1_ref,
             s_ref, t_ref):
    xu = xu_ref[...]
    xi = xi_ref[...]
    s_ref[0] = jnp.dot(xu, wsu_ref[...], preferred_element_type=jnp.float32)
    s_ref[1] = jnp.dot(xi, wsi_ref[...], preferred_element_type=jnp.float32)
    t_ref[0] = jnp.dot(xi, wt0_ref[...], preferred_element_type=jnp.float32)
    t_ref[1] = jnp.dot(xu, wt1_ref[...], preferred_element_type=jnp.float32)


def _epilogue_body(t_ref, s_ref, c_ref, out_ref):
    c0 = jnp.maximum(c_ref[0][:, 0:1], 1.0)
    c1 = jnp.maximum(c_ref[1][:, 0:1], 1.0)
    out_ref[0] = jnp.maximum(t_ref[1] + s_ref[1] / c1, 0.0)
    out_ref[1] = jnp.maximum(t_ref[0] + s_ref[0] / c0, 0.0)


def _sc_agg(s_flat, srcx_all, dst_all, sums_out,
            idx_sg, idx_dw, buf0, buf1, buf2, acc,
            gs0, gs1, gs2, ss0, ss1, ss2):
    cid = lax.axis_index("c")
    sid = lax.axis_index("s")
    bufs = (buf0, buf1, buf2)
    gsem = (gs0, gs1, gs2)
    ssem = (ss0, ss1, ss2)
    ebase = sid * GPT
    wbase = sid * ROWS_W

    def fill(ref, val):
        def body(i, carry):
            ref[i, pl.ds(0, 16)] = jnp.full((16,), val, jnp.float32)
            ref[i, pl.ds(16, 16)] = jnp.full((16,), val, jnp.float32)
            return carry
        lax.fori_loop(0, G, body, 0)

    def zero_acc(zbuf):
        base = sid * ROWS_Z
        nfull = ROWS_Z // G

        def body(j, carry):
            pltpu.async_copy(zbuf, acc.at[pl.ds(base + G * j, G), :], ss0)
            return carry
        lax.fori_loop(0, nfull, body, 0)
        rem = ROWS_Z - nfull * G
        pltpu.sync_copy(zbuf.at[pl.ds(0, rem), :],
                        acc.at[pl.ds(base + nfull * G, rem), :])

        def drain(j, carry):
            pltpu.make_async_copy(zbuf, acc.at[pl.ds(base, G), :], ss0).wait()
            return carry
        lax.fori_loop(0, nfull, drain, 0)

    def drain_scatter(s):
        pltpu.make_async_copy(
            bufs[s], acc.at[idx_dw.at[pl.ds(0, G)]], ssem[s]).wait()

    def chunk_pipeline(r, c):
        tab = s_flat.at[r]

        def issue_gather(h, u, s):
            pltpu.sync_copy(
                srcx_all.at[r, c, pl.ds(ebase + (h * SW + u) * G, G)],
                idx_sg.at[s])
            return pltpu.async_copy(tab.at[idx_sg.at[s]], bufs[s], gsem[s])

        def sw_body(h, carry):
            @pl.when(h > 0)
            def _():
                for s in range(3):
                    drain_scatter(s)
            dwd = pltpu.async_copy(
                dst_all.at[r, pl.ds(ebase + h * (G * SW), G * SW)], idx_dw,
                ss0)
            gd = [None] * SW
            for s in range(3):
                gd[s] = issue_gather(h, s, s)
            dwd.wait()
            for u in range(SW):
                s = u % 3
                gd[u].wait()
                sd = pltpu.async_copy(
                    bufs[s], acc.at[idx_dw.at[pl.ds(u * G, G)]], ssem[s],
                    add=True)
                if u + 3 < SW:
                    sd.wait()
                    gd[u + 3] = issue_gather(h, u + 3, s)
            return carry
        lax.fori_loop(0, NSW, sw_body, 0)
        for s in range(3):
            drain_scatter(s)

    for r in range(2):
        for p in range(2):
            fill(buf0, 0.0)
            zero_acc(buf0)
            plsc.subcore_barrier()
            for cc in range(NC):
                @pl.when(cid == cc)
                def _(r=r, c=NC * cc + p):
                    chunk_pipeline(r, c)
            plsc.subcore_barrier()
            for cc in range(NC):
                @pl.when(cid == cc)
                def _(r=r, c=NC * cc + p):
                    pltpu.sync_copy(
                        acc.at[pl.ds(wbase, ROWS_W), :],
                        sums_out.at[r, pl.ds(wbase, ROWS_W), pl.ds(CW * c, CW)])
            plsc.subcore_barrier()

def _sc_count(dst_all, cnt_out, idx_dw, buf0, buf1, acc, ss0):
    cid = lax.axis_index("c")
    sid = lax.axis_index("s")
    ebase = sid * GPT
    wbase = sid * ROWS_W

    def fill(ref, val):
        def body(i, carry):
            ref[i, pl.ds(0, 16)] = jnp.full((16,), val, jnp.float32)
            ref[i, pl.ds(16, 16)] = jnp.full((16,), val, jnp.float32)
            return carry
        lax.fori_loop(0, G, body, 0)

    def zero_acc(zbuf):
        base = sid * ROWS_Z
        nfull = ROWS_Z // G

        def body(j, carry):
            pltpu.async_copy(zbuf, acc.at[pl.ds(base + G * j, G), :], ss0)
            return carry
        lax.fori_loop(0, nfull, body, 0)
        rem = ROWS_Z - nfull * G
        pltpu.sync_copy(zbuf.at[pl.ds(0, rem), :],
                        acc.at[pl.ds(base + nfull * G, rem), :])

        def drain(j, carry):
            pltpu.make_async_copy(zbuf, acc.at[pl.ds(base, G), :], ss0).wait()
            return carry
        lax.fori_loop(0, nfull, drain, 0)

    fill(buf1, 0.0)
    zero_acc(buf1)
    fill(buf0, 1.0)
    plsc.subcore_barrier()
    for cc in range(NC):
        @pl.when(cid == cc)
        def _(cc=cc):
            def sw_body(h, carry):
                pltpu.sync_copy(
                    dst_all.at[cc, pl.ds(ebase + h * (G * SW), G * SW)],
                    idx_dw)
                descs = [
                    pltpu.async_copy(
                        buf0, acc.at[idx_dw.at[pl.ds(u * G, G)]], ss0,
                        add=True)
                    for u in range(SW)
                ]
                for d in descs:
                    d.wait()
                return carry
            lax.fori_loop(0, NSW, sw_body, 0)
    plsc.subcore_barrier()
    for cc in range(NC):
        @pl.when(cid == cc)
        def _(cc=cc):
            pltpu.sync_copy(acc.at[pl.ds(wbase, ROWS_W), :],
                            cnt_out.at[cc, pl.ds(wbase, ROWS_W), pl.ds(0, CW)])


@functools.cache
def _sc_agg_call():
    return pl.kernel(
        _sc_agg,
        out_type=jax.ShapeDtypeStruct((2, N_NODES, D), jnp.float32),
        mesh=plsc.VectorSubcoreMesh(core_axis_name="c", subcore_axis_name="s",
                                    num_cores=NC, num_subcores=NS),
        scratch_types=[
            pltpu.VMEM((3, G), jnp.int32),
            pltpu.VMEM((G * SW,), jnp.int32),
            pltpu.VMEM((G, CW), jnp.float32),
            pltpu.VMEM((G, CW), jnp.float32),
            pltpu.VMEM((G, CW), jnp.float32),
            pltpu.VMEM_SHARED((NACC, CW), jnp.float32),
            pltpu.SemaphoreType.DMA,
            pltpu.SemaphoreType.DMA,
            pltpu.SemaphoreType.DMA,
            pltpu.SemaphoreType.DMA,
            pltpu.SemaphoreType.DMA,
            pltpu.SemaphoreType.DMA,
        ],
        compiler_params=pltpu.CompilerParams(use_tc_tiling_on_sc=False),
    )


@functools.cache
def _sc_count_call():
    return pl.kernel(
        _sc_count,
        out_type=jax.ShapeDtypeStruct((2, N_NODES, D), jnp.float32),
        mesh=plsc.VectorSubcoreMesh(core_axis_name="c", subcore_axis_name="s",
                                    num_cores=NC, num_subcores=NS),
        scratch_types=[
            pltpu.VMEM((G * SW,), jnp.int32),
            pltpu.VMEM((G, CW), jnp.float32),
            pltpu.VMEM((G, CW), jnp.float32),
            pltpu.VMEM_SHARED((NACC, CW), jnp.float32),
            pltpu.SemaphoreType.DMA,
        ],
        compiler_params=pltpu.CompilerParams(use_tc_tiling_on_sc=False),
    )


def _prep_edges(edge):
    pad = EPAD - edge.shape[1]
    padr = jnp.arange(pad, dtype=jnp.int32)
    src = jnp.concatenate([edge[0], padr % 2048])
    srcx = jnp.stack([src * NCHUNK + c for c in range(NCHUNK)])
    dst = jnp.concatenate([edge[1], N_NODES + (padr % NPAD)])
    return srcx, dst


def kernel(x_user, x_item, edge_rates, edge_rated_by,
           W_rates_src, W_rates_tgt, W_rb_src, W_rb_tgt):
    n_blocks = N_NODES // BM

    s_all, t_all = pl.pallas_call(
        _mm_body,
        grid=(n_blocks,),
        in_specs=[
            pl.BlockSpec((BM, D), lambda i: (i, 0)),
            pl.BlockSpec((BM, D), lambda i: (i, 0)),
            pl.BlockSpec((D, D), lambda i: (0, 0)),
            pl.BlockSpec((D, D), lambda i: (0, 0)),
            pl.BlockSpec((D, D), lambda i: (0, 0)),
            pl.BlockSpec((D, D), lambda i: (0, 0)),
        ],
        out_specs=[
            pl.BlockSpec((2, BM, D), lambda i: (0, i, 0)),
            pl.BlockSpec((2, BM, D), lambda i: (0, i, 0)),
        ],
        out_shape=[
            jax.ShapeDtypeStruct((2, N_NODES, D), jnp.float32),
            jax.ShapeDtypeStruct((2, N_NODES, D), jnp.float32),
        ],
    )(x_user, x_item, W_rates_src, W_rb_src, W_rates_tgt, W_rb_tgt)

    s_flat = s_all.reshape(2, NCHUNK * N_NODES, CW)

    srcx0, dst0 = _prep_edges(edge_rates)
    srcx1, dst1 = _prep_edges(edge_rated_by)
    srcx_all = jnp.stack([srcx0, srcx1])
    dst_all = jnp.stack([dst0, dst1])

    cnt = _sc_count_call()(dst_all)
    sums = _sc_agg_call()(s_flat, srcx_all, dst_all)

    out = pl.pallas_call(
        _epilogue_body,
        grid=(n_blocks,),
        in_specs=[
            pl.BlockSpec((2, BM, D), lambda i: (0, i, 0)),
            pl.BlockSpec((2, BM, D), lambda i: (0, i, 0)),
            pl.BlockSpec((2, BM, D), lambda i: (0, i, 0)),
        ],
        out_specs=pl.BlockSpec((2, BM, D), lambda i: (0, i, 0)),
        out_shape=jax.ShapeDtypeStruct((2, N_NODES, D), jnp.float32),
    )(t_all, sums, cnt)

    return out
